# Initial kernel scaffold; baseline (speedup 1.0000x reference)
import functools

import jax
import jax.numpy as jnp
from jax import lax
from jax.experimental import pallas as pl
from jax.experimental.pallas import tpu as pltpu
from jax.experimental.pallas import tpu_sc as plsc

NUM_CORES = 2
NUM_SUBCORES = 16
NUM_WORKERS = NUM_CORES * NUM_SUBCORES

EMB_DIM = 32
CHUNK = 1024


@functools.partial(jax.jit, static_argnums=(2,))
def _sc_gather(ids_flat, table, n):
    b_per_w = n // NUM_WORKERS
    n_chunks = b_per_w // CHUNK
    mesh = plsc.VectorSubcoreMesh(
        core_axis_name="c", subcore_axis_name="s",
        num_cores=NUM_CORES, num_subcores=NUM_SUBCORES)

    @functools.partial(
        pl.kernel,
        out_type=jax.ShapeDtypeStruct((n, EMB_DIM), jnp.float32),
        mesh=mesh,
        scratch_types=[
            pltpu.VMEM((CHUNK,), jnp.int32),
            pltpu.VMEM((CHUNK, EMB_DIM), jnp.float32),
            pltpu.SemaphoreType.DMA,
        ],
    )
    def body(ids_hbm, tab_hbm, out_hbm, idx_v, rows_v, sem):
        wid = lax.axis_index("s") * NUM_CORES + lax.axis_index("c")
        base = wid * b_per_w

        def chunk_step(g, carry):
            off = base + g * CHUNK
            pltpu.sync_copy(ids_hbm.at[pl.ds(off, CHUNK)], idx_v)
            pltpu.async_copy(tab_hbm.at[idx_v], rows_v, sem).wait()
            pltpu.sync_copy(rows_v, out_hbm.at[pl.ds(off, CHUNK)])
            return carry

        lax.fori_loop(0, n_chunks, chunk_step, 0, unroll=False)

    return body(ids_flat, table)


def kernel(token_ids, embedding_matrix):
    flat = token_ids.reshape(-1)
    out = _sc_gather(flat, embedding_matrix, flat.shape[0])
    return out.reshape(token_ids.shape + (EMB_DIM,))


# speedup vs baseline: 1.0944x; 1.0944x over previous
import functools

import jax
import jax.numpy as jnp
from jax import lax
from jax.experimental import pallas as pl
from jax.experimental.pallas import tpu as pltpu
from jax.experimental.pallas import tpu_sc as plsc

NUM_CORES = 2
NUM_SUBCORES = 16
NUM_WORKERS = NUM_CORES * NUM_SUBCORES

EMB_DIM = 32
CHUNK = 1024


@functools.partial(jax.jit, static_argnums=(2,))
def _sc_gather(ids_flat, table, n):
    b_per_w = n // NUM_WORKERS
    n_chunks = b_per_w // CHUNK
    mesh = plsc.VectorSubcoreMesh(
        core_axis_name="c", subcore_axis_name="s",
        num_cores=NUM_CORES, num_subcores=NUM_SUBCORES)

    @functools.partial(
        pl.kernel,
        out_type=jax.ShapeDtypeStruct((n, EMB_DIM), jnp.float32),
        mesh=mesh,
        scratch_types=[
            pltpu.VMEM((CHUNK,), jnp.int32),
            pltpu.VMEM((CHUNK, EMB_DIM), jnp.float32),
            pltpu.SemaphoreType.DMA,
        ],
        compiler_params=pltpu.CompilerParams(use_tc_tiling_on_sc=False),
    )
    def body(ids_hbm, tab_hbm, out_hbm, idx_v, rows_v, sem):
        wid = lax.axis_index("s") * NUM_CORES + lax.axis_index("c")
        base = wid * b_per_w

        def chunk_step(g, carry):
            off = base + g * CHUNK
            pltpu.sync_copy(ids_hbm.at[pl.ds(off, CHUNK)], idx_v)
            pltpu.async_copy(tab_hbm.at[idx_v], rows_v, sem).wait()
            pltpu.sync_copy(rows_v, out_hbm.at[pl.ds(off, CHUNK)])
            return carry

        lax.fori_loop(0, n_chunks, chunk_step, 0, unroll=False)

    return body(ids_flat, table)


def kernel(token_ids, embedding_matrix):
    flat = token_ids.reshape(-1)
    out = _sc_gather(flat, embedding_matrix, flat.shape[0])
    return out.reshape(token_ids.shape + (EMB_DIM,))


# speedup vs baseline: 1.1102x; 1.0144x over previous
import functools

import jax
import jax.numpy as jnp
from jax import lax
from jax.experimental import pallas as pl
from jax.experimental.pallas import tpu as pltpu
from jax.experimental.pallas import tpu_sc as plsc

NUM_CORES = 2
NUM_SUBCORES = 16
NUM_WORKERS = NUM_CORES * NUM_SUBCORES

EMB_DIM = 32
CHUNK = 1600
NBUF = 2


@functools.partial(jax.jit, static_argnums=(2,))
def _sc_gather(ids_flat, table, n):
    b_per_w = n // NUM_WORKERS
    n_chunks = b_per_w // CHUNK
    assert b_per_w % CHUNK == 0
    mesh = plsc.VectorSubcoreMesh(
        core_axis_name="c", subcore_axis_name="s",
        num_cores=NUM_CORES, num_subcores=NUM_SUBCORES)

    @functools.partial(
        pl.kernel,
        out_type=jax.ShapeDtypeStruct((n, EMB_DIM), jnp.float32),
        mesh=mesh,
        scratch_types=[
            pltpu.VMEM((NBUF, CHUNK), jnp.int32),
            pltpu.VMEM((NBUF, CHUNK, EMB_DIM), jnp.float32),
            [pltpu.SemaphoreType.DMA] * NBUF,
            [pltpu.SemaphoreType.DMA] * NBUF,
            [pltpu.SemaphoreType.DMA] * NBUF,
        ],
        compiler_params=pltpu.CompilerParams(use_tc_tiling_on_sc=False),
    )
    def body(ids_hbm, tab_hbm, out_hbm, idx_v, rows_v, isem, gsem, ssem):
        wid = lax.axis_index("s") * NUM_CORES + lax.axis_index("c")
        base = wid * b_per_w

        for g in range(min(NBUF, n_chunks)):
            pltpu.async_copy(
                ids_hbm.at[pl.ds(base + g * CHUNK, CHUNK)], idx_v.at[g],
                isem[g])

        for g in range(n_chunks):
            b = g % NBUF
            off = base + g * CHUNK
            pltpu.make_async_copy(
                ids_hbm.at[pl.ds(off, CHUNK)], idx_v.at[b], isem[b]).wait()
            if g >= NBUF:
                pltpu.make_async_copy(
                    rows_v.at[b],
                    out_hbm.at[pl.ds(base + (g - NBUF) * CHUNK, CHUNK)],
                    ssem[b]).wait()
            pltpu.async_copy(tab_hbm.at[idx_v.at[b]], rows_v.at[b],
                             gsem[b]).wait()
            pltpu.async_copy(
                rows_v.at[b], out_hbm.at[pl.ds(off, CHUNK)], ssem[b])
            if g + NBUF < n_chunks:
                pltpu.async_copy(
                    ids_hbm.at[pl.ds(base + (g + NBUF) * CHUNK, CHUNK)],
                    idx_v.at[b], isem[b])

        for g in range(max(0, n_chunks - NBUF), n_chunks):
            b = g % NBUF
            pltpu.make_async_copy(
                rows_v.at[b], out_hbm.at[pl.ds(base + g * CHUNK, CHUNK)],
                ssem[b]).wait()

    return body(ids_flat, table)


def kernel(token_ids, embedding_matrix):
    flat = token_ids.reshape(-1)
    out = _sc_gather(flat, embedding_matrix, flat.shape[0])
    return out.reshape(token_ids.shape + (EMB_DIM,))


# speedup vs baseline: 1.5429x; 1.3898x over previous
import functools

import jax
import jax.numpy as jnp
from jax import lax
from jax.experimental import pallas as pl
from jax.experimental.pallas import tpu as pltpu
from jax.experimental.pallas import tpu_sc as plsc

NUM_CORES = 2
NUM_SUBCORES = 16
NUM_WORKERS = NUM_CORES * NUM_SUBCORES

NI = 16384
NJ = 50
D = 32
SUBI = 16
CHUNK = SUBI * NJ
I_PER_W = NI // NUM_WORKERS
N_CHUNKS = I_PER_W // SUBI


def _gather_body(ids_hbm, tab_hbm, out_hbm, idx_v, rows_v, w_v, isem, gsem,
                 ssem):
    wid = lax.axis_index("s") * NUM_CORES + lax.axis_index("c")
    i_base = wid * I_PER_W

    lane = lax.iota(jnp.int32, 16)

    def start_idx(g, b):
        pltpu.async_copy(
            ids_hbm.at[pl.ds((i_base + g * SUBI) * NJ, CHUNK)], idx_v.at[b],
            isem[b])

    def wait_idx(g, b):
        pltpu.make_async_copy(
            ids_hbm.at[pl.ds((i_base + g * SUBI) * NJ, CHUNK)], idx_v.at[b],
            isem[b]).wait()

    def start_gather(b):
        pltpu.async_copy(tab_hbm.at[idx_v.at[b]], rows_v.at[b], gsem[b])

    def wait_gather(b):
        pltpu.make_async_copy(tab_hbm.at[idx_v.at[b]], rows_v.at[b],
                              gsem[b]).wait()

    def out_slab(g):
        i0 = i_base + g * SUBI
        return out_hbm.at[:, :, i0 // 128, :, pl.ds(i0 % 128, SUBI)]

    def start_store(g, b):
        pltpu.async_copy(w_v.at[b], out_slab(g), ssem[b])

    def wait_store(g, b):
        pltpu.make_async_copy(w_v.at[b], out_slab(g), ssem[b]).wait()

    def transpose_chunk(b):
        rows = rows_v.at[b]
        w = w_v.at[b]

        def per_j(j, carry):
            row_idx = lane * NJ + j
            for d in range(D):
                vals = plsc.load_gather(
                    rows, [row_idx, jnp.full((16,), d, jnp.int32)])
                w[j, d // 8, d % 8, :] = vals
            return carry

        lax.fori_loop(0, NJ, per_j, 0, unroll=False)

    start_idx(0, 0)
    start_idx(1, 1)
    wait_idx(0, 0)
    start_gather(0)

    def pair(gg, carry):
        g0 = gg * 2
        for b in range(2):
            g = g0 + b
            nb = 1 - b
            wait_gather(b)
            @pl.when(g + 1 < N_CHUNKS)
            def _():
                wait_idx(g + 1, nb)
                start_gather(nb)

            @pl.when(g >= 2)
            def _():
                wait_store(g - 2, b)

            transpose_chunk(b)
            start_store(g, b)

            @pl.when(g + 2 < N_CHUNKS)
            def _():
                start_idx(g + 2, b)
        return carry

    lax.fori_loop(0, N_CHUNKS // 2, pair, 0, unroll=False)

    for g in range(N_CHUNKS - 2, N_CHUNKS):
        wait_store(g, g % 2)


@jax.jit
def _sc_gather(ids_flat, table):
    mesh = plsc.VectorSubcoreMesh(
        core_axis_name="c", subcore_axis_name="s",
        num_cores=NUM_CORES, num_subcores=NUM_SUBCORES)
    return pl.kernel(
        _gather_body,
        out_type=jax.ShapeDtypeStruct((NJ, D // 8, NI // 128, 8, 128),
                                      jnp.float32),
        mesh=mesh,
        scratch_types=[
            pltpu.VMEM((2, CHUNK), jnp.int32),
            pltpu.VMEM((2, CHUNK, D), jnp.float32),
            pltpu.VMEM((2, NJ, D // 8, 8, SUBI), jnp.float32),
            [pltpu.SemaphoreType.DMA] * 2,
            [pltpu.SemaphoreType.DMA] * 2,
            [pltpu.SemaphoreType.DMA] * 2,
        ],
        compiler_params=pltpu.CompilerParams(use_tc_tiling_on_sc=False,
                                             needs_layout_passes=False),
    )(ids_flat, table)


def kernel(token_ids, embedding_matrix):
    ids_flat = token_ids.reshape(-1)
    out5 = _sc_gather(ids_flat, embedding_matrix)
    return out5.transpose(2, 4, 0, 1, 3).reshape(NI, NJ, D)


# speedup vs baseline: 2.3722x; 1.5375x over previous
import functools

import jax
import jax.numpy as jnp
from jax import lax
from jax.experimental import pallas as pl
from jax.experimental.pallas import tpu as pltpu
from jax.experimental.pallas import tpu_sc as plsc

NUM_CORES = 2
NUM_SUBCORES = 16
NUM_WORKERS = NUM_CORES * NUM_SUBCORES

NI = 16384
NJ = 50
D = 32
SUBI = 16
CHUNK = SUBI * NJ
I_PER_W = NI // NUM_WORKERS
N_CHUNKS = I_PER_W // SUBI


def _gather_body(ids_hbm, tab_hbm, out_hbm, idx_v, rows_v, w_v, isem, gsem,
                 ssem):
    wid = lax.axis_index("s") * NUM_CORES + lax.axis_index("c")
    i_base = wid * I_PER_W

    lane = lax.iota(jnp.int32, 16)
    dhalf = []
    for d0 in (0, 16):
        d_all = lane + d0
        dhalf.append((lax.shift_right_logical(d_all, 3),
                      lax.bitwise_and(d_all, 7)))

    def start_idx(g, b):
        pltpu.async_copy(
            ids_hbm.at[pl.ds((i_base + g * SUBI) * NJ, CHUNK)], idx_v.at[b],
            isem[b])

    def wait_idx(g, b):
        pltpu.make_async_copy(
            ids_hbm.at[pl.ds((i_base + g * SUBI) * NJ, CHUNK)], idx_v.at[b],
            isem[b]).wait()

    def start_gather(b):
        pltpu.async_copy(tab_hbm.at[idx_v.at[b]], rows_v.at[b], gsem[b])

    def wait_gather(b):
        pltpu.make_async_copy(tab_hbm.at[idx_v.at[b]], rows_v.at[b],
                              gsem[b]).wait()

    def _store_pair(g, b, j):
        i0 = i_base + g * SUBI
        return (w_v.at[b, j, :, :, pl.ds(0, SUBI)],
                out_hbm.at[j, :, i0 // 128, :, pl.ds(i0 % 128, SUBI)])

    def start_store(g, b):
        def per_j(j, carry):
            src, dst = _store_pair(g, b, j)
            pltpu.async_copy(src, dst, ssem[b])
            return carry
        lax.fori_loop(0, NJ, per_j, 0, unroll=False)

    def wait_store(g, b):
        def per_j(j, carry):
            src, dst = _store_pair(g, b, j)
            pltpu.make_async_copy(src, dst, ssem[b]).wait()
            return carry
        lax.fori_loop(0, NJ, per_j, 0, unroll=False)

    def transpose_chunk(b):
        rows = rows_v.at[b]
        w = w_v.at[b]

        def per_iloc(i_loc, c0):
            iv = jnp.full((16,), i_loc, jnp.int32)
            kbase = i_loc * NJ

            def per_j(j, c1):
                wj = w.at[j]
                k = kbase + j
                for h, (dhi, dlo) in enumerate(dhalf):
                    vals = rows[k, pl.ds(h * 16, 16)]
                    plsc.store_scatter(wj, [dhi, dlo, iv], vals)
                return c1

            lax.fori_loop(0, NJ, per_j, 0, unroll=10)
            return c0

        lax.fori_loop(0, SUBI, per_iloc, 0, unroll=False)

    start_idx(0, 0)
    start_idx(1, 1)
    wait_idx(0, 0)
    start_gather(0)

    def pair(gg, carry):
        g0 = gg * 2
        for b in range(2):
            g = g0 + b
            nb = 1 - b
            wait_gather(b)
            @pl.when(g + 1 < N_CHUNKS)
            def _():
                wait_idx(g + 1, nb)
                start_gather(nb)

            @pl.when(g >= 2)
            def _():
                wait_store(g - 2, b)

            transpose_chunk(b)
            start_store(g, b)

            @pl.when(g + 2 < N_CHUNKS)
            def _():
                start_idx(g + 2, b)
        return carry

    lax.fori_loop(0, N_CHUNKS // 2, pair, 0, unroll=False)

    for g in range(N_CHUNKS - 2, N_CHUNKS):
        wait_store(g, g % 2)


@jax.jit
def _sc_gather(ids_flat, table):
    mesh = plsc.VectorSubcoreMesh(
        core_axis_name="c", subcore_axis_name="s",
        num_cores=NUM_CORES, num_subcores=NUM_SUBCORES)
    return pl.kernel(
        _gather_body,
        out_type=jax.ShapeDtypeStruct((NJ, D // 8, NI // 128, 8, 128),
                                      jnp.float32),
        mesh=mesh,
        scratch_types=[
            pltpu.VMEM((2, CHUNK), jnp.int32),
            pltpu.VMEM((2, CHUNK, D), jnp.float32),
            pltpu.VMEM((2, NJ, D // 8, 8, SUBI + 1), jnp.float32),
            [pltpu.SemaphoreType.DMA] * 2,
            [pltpu.SemaphoreType.DMA] * 2,
            [pltpu.SemaphoreType.DMA] * 2,
        ],
        compiler_params=pltpu.CompilerParams(use_tc_tiling_on_sc=False,
                                             needs_layout_passes=False),
    )(ids_flat, table)


def kernel(token_ids, embedding_matrix):
    ids_flat = token_ids.reshape(-1)
    out5 = _sc_gather(ids_flat, embedding_matrix)
    return out5.transpose(2, 4, 0, 1, 3).reshape(NI, NJ, D)


# speedup vs baseline: 2.3726x; 1.0002x over previous
import functools

import jax
import jax.numpy as jnp
from jax import lax
from jax.experimental import pallas as pl
from jax.experimental.pallas import tpu as pltpu
from jax.experimental.pallas import tpu_sc as plsc

NUM_CORES = 2
NUM_SUBCORES = 16
NUM_WORKERS = NUM_CORES * NUM_SUBCORES

NI = 16384
NJ = 50
D = 32
SUBI = 16
CHUNK = SUBI * NJ
I_PER_W = NI // NUM_WORKERS
N_CHUNKS = I_PER_W // SUBI


def _gather_body(ids_hbm, tab_hbm, out_hbm, idx_v, rows_v, w_v, isem, gsem,
                 ssem):
    wid = lax.axis_index("s") * NUM_CORES + lax.axis_index("c")
    i_base = wid * I_PER_W

    lane = lax.iota(jnp.int32, 16)
    dhalf = []
    for d0 in (0, 16):
        d_all = lane + d0
        dhalf.append((lax.shift_right_logical(d_all, 3),
                      lax.bitwise_and(d_all, 7)))

    def start_idx(g, b):
        pltpu.async_copy(
            ids_hbm.at[pl.ds((i_base + g * SUBI) * NJ, CHUNK)], idx_v.at[b],
            isem[b])

    def wait_idx(g, b):
        pltpu.make_async_copy(
            ids_hbm.at[pl.ds((i_base + g * SUBI) * NJ, CHUNK)], idx_v.at[b],
            isem[b]).wait()

    def start_gather(b):
        pltpu.async_copy(tab_hbm.at[idx_v.at[b]], rows_v.at[b], gsem[b])

    def wait_gather(b):
        pltpu.make_async_copy(tab_hbm.at[idx_v.at[b]], rows_v.at[b],
                              gsem[b]).wait()

    def _store_pair(g, b, j):
        i0 = i_base + g * SUBI
        return (w_v.at[b, j, :, :, pl.ds(0, SUBI)],
                out_hbm.at[j, :, i0 // 128, :, pl.ds(i0 % 128, SUBI)])

    def start_store(g, b):
        def per_j(j, carry):
            src, dst = _store_pair(g, b, j)
            pltpu.async_copy(src, dst, ssem[b])
            return carry
        lax.fori_loop(0, NJ, per_j, 0, unroll=False)

    def wait_store(g, b):
        def per_j(j, carry):
            src, dst = _store_pair(g, b, j)
            pltpu.make_async_copy(src, dst, ssem[b]).wait()
            return carry
        lax.fori_loop(0, NJ, per_j, 0, unroll=False)

    def transpose_chunk(b):
        rows = rows_v.at[b]
        w = w_v.at[b]

        def per_iloc(i_loc, c0):
            iv = jnp.full((16,), i_loc, jnp.int32)
            kbase = i_loc * NJ

            def per_j(j, c1):
                wj = w.at[j]
                k = kbase + j
                for h, (dhi, dlo) in enumerate(dhalf):
                    vals = rows[k, pl.ds(h * 16, 16)]
                    plsc.store_scatter(wj, [dhi, dlo, iv], vals)
                return c1

            lax.fori_loop(0, NJ, per_j, 0, unroll=10)
            return c0

        lax.fori_loop(0, SUBI, per_iloc, 0, unroll=False)

    start_idx(0, 0)
    start_idx(1, 1)
    wait_idx(0, 0)
    start_gather(0)

    def pair(gg, carry):
        g0 = gg * 2
        for b in range(2):
            g = g0 + b
            nb = 1 - b
            wait_gather(b)
            @pl.when(g + 1 < N_CHUNKS)
            def _():
                wait_idx(g + 1, nb)
                start_gather(nb)

            @pl.when(g >= 2)
            def _():
                wait_store(g - 2, b)

            transpose_chunk(b)
            start_store(g, b)

            @pl.when(g + 2 < N_CHUNKS)
            def _():
                start_idx(g + 2, b)
        return carry

    lax.fori_loop(0, N_CHUNKS // 2, pair, 0, unroll=False)

    for g in range(N_CHUNKS - 2, N_CHUNKS):
        wait_store(g, g % 2)


@jax.jit
def _sc_gather(ids_flat, table):
    mesh = plsc.VectorSubcoreMesh(
        core_axis_name="c", subcore_axis_name="s",
        num_cores=NUM_CORES, num_subcores=NUM_SUBCORES)
    return pl.kernel(
        _gather_body,
        out_type=jax.ShapeDtypeStruct((NJ, D // 8, NI // 128, 8, 128),
                                      jnp.float32),
        mesh=mesh,
        scratch_types=[
            pltpu.VMEM((2, CHUNK), jnp.int32),
            pltpu.VMEM((2, CHUNK, D), jnp.float32),
            pltpu.VMEM((2, NJ, D // 8, 8, SUBI + 1), jnp.float32),
            [pltpu.SemaphoreType.DMA] * 2,
            [pltpu.SemaphoreType.DMA] * 2,
            [pltpu.SemaphoreType.DMA] * 2,
        ],
        compiler_params=pltpu.CompilerParams(use_tc_tiling_on_sc=False,
                                             needs_layout_passes=False,
                                             disable_bounds_checks=True),
    )(ids_flat, table)


def kernel(token_ids, embedding_matrix):
    ids_flat = token_ids.reshape(-1)
    out5 = _sc_gather(ids_flat, embedding_matrix)
    return out5.transpose(2, 4, 0, 1, 3).reshape(NI, NJ, D)


# speedup vs baseline: 2.8064x; 1.1828x over previous
import functools

import jax
import jax.numpy as jnp
from jax import lax
from jax.experimental import pallas as pl
from jax.experimental.pallas import tpu as pltpu
from jax.experimental.pallas import tpu_sc as plsc

NUM_CORES = 2
NUM_SUBCORES = 16
NUM_WORKERS = NUM_CORES * NUM_SUBCORES

NI = 16384
NJ = 50
D = 32
SUBI = 16
CHUNK = SUBI * NJ
I_PER_W = NI // NUM_WORKERS
N_CHUNKS = I_PER_W // SUBI


def _gather_body(ids_hbm, tab_hbm, out_hbm, idx_v, rows_v, w_v, isem, gsem,
                 ssem):
    wid = lax.axis_index("s") * NUM_CORES + lax.axis_index("c")
    i_base = wid * I_PER_W

    lane = lax.iota(jnp.int32, 16)
    dhalf = []
    for d0 in (0, 16):
        d_all = lane + d0
        dhalf.append((lax.shift_right_logical(d_all, 3),
                      lax.bitwise_and(d_all, 7)))

    def start_idx(g, b):
        pltpu.async_copy(
            ids_hbm.at[pl.ds((i_base + g * SUBI) * NJ, CHUNK)], idx_v.at[b],
            isem[b])

    def wait_idx(g, b):
        pltpu.make_async_copy(
            ids_hbm.at[pl.ds((i_base + g * SUBI) * NJ, CHUNK)], idx_v.at[b],
            isem[b]).wait()

    def start_gather(b):
        pltpu.async_copy(tab_hbm.at[idx_v.at[b]], rows_v.at[b], gsem[b])

    def wait_gather(b):
        pltpu.make_async_copy(tab_hbm.at[idx_v.at[b]], rows_v.at[b],
                              gsem[b]).wait()

    def _store_pair(g, b, j):
        i0 = i_base + g * SUBI
        return (w_v.at[b, j, :, :, pl.ds(0, SUBI)],
                out_hbm.at[j, :, i0 // 128, :, pl.ds(i0 % 128, SUBI)])

    def start_store(g, b):
        def per_j(j, carry):
            src, dst = _store_pair(g, b, j)
            pltpu.async_copy(src, dst, ssem[b])
            return carry
        lax.fori_loop(0, NJ, per_j, 0, unroll=False)

    def wait_store(g, b):
        def per_j(j, carry):
            src, dst = _store_pair(g, b, j)
            pltpu.make_async_copy(src, dst, ssem[b]).wait()
            return carry
        lax.fori_loop(0, NJ, per_j, 0, unroll=False)

    def transpose_chunk(b):
        rows = rows_v.at[b]
        w = w_v.at[b]

        def per_iloc(i_loc, c0):
            iv = jnp.full((16,), i_loc, jnp.int32)
            kbase = i_loc * NJ

            def per_j(j, c1):
                wj = w.at[j]
                k = kbase + j
                for h, (dhi, dlo) in enumerate(dhalf):
                    vals = rows[k, pl.ds(h * 16, 16)]
                    plsc.store_scatter(wj, [dhi, dlo, iv], vals)
                return c1

            lax.fori_loop(0, NJ, per_j, 0, unroll=10)
            return c0

        lax.fori_loop(0, SUBI, per_iloc, 0, unroll=False)

    start_idx(0, 0)
    start_idx(1, 1)
    wait_idx(0, 0)
    start_gather(0)

    def pair(gg, carry):
        g0 = gg * 2
        for b in range(2):
            g = g0 + b
            nb = 1 - b
            wait_gather(b)
            @pl.when(g + 1 < N_CHUNKS)
            def _():
                wait_idx(g + 1, nb)
                start_gather(nb)

            @pl.when(g >= 2)
            def _():
                wait_store(g - 2, b)

            transpose_chunk(b)
            start_store(g, b)

            @pl.when(g + 2 < N_CHUNKS)
            def _():
                start_idx(g + 2, b)
        return carry

    lax.fori_loop(0, N_CHUNKS // 2, pair, 0, unroll=False)

    for g in range(N_CHUNKS - 2, N_CHUNKS):
        wait_store(g, g % 2)


@jax.jit
def _sc_gather(ids_flat, table):
    mesh = plsc.VectorSubcoreMesh(
        core_axis_name="c", subcore_axis_name="s",
        num_cores=NUM_CORES, num_subcores=NUM_SUBCORES)
    return pl.kernel(
        _gather_body,
        out_type=jax.ShapeDtypeStruct((NJ, D // 8, NI // 128, 8, 128),
                                      jnp.float32),
        mesh=mesh,
        scratch_types=[
            pltpu.VMEM((2, CHUNK), jnp.int32),
            pltpu.VMEM((2, CHUNK, D), jnp.float32),
            pltpu.VMEM((2, NJ, D // 8, 8, SUBI + 1), jnp.float32),
            [pltpu.SemaphoreType.DMA] * 2,
            [pltpu.SemaphoreType.DMA] * 2,
            [pltpu.SemaphoreType.DMA] * 2,
        ],
        compiler_params=pltpu.CompilerParams(use_tc_tiling_on_sc=False,
                                             needs_layout_passes=False,
                                             disable_bounds_checks=True),
    )(ids_flat, table)


_BJ = 2048


def _tr_body(in_ref, out_ref):
    y = in_ref[...].T.reshape(_BJ, 4, 32)
    for c in range(4):
        out_ref[:, c * 32:(c + 1) * 32] = y[:, c, :]


@jax.jit
def _tc_transpose(tab_t):
    num_emb = tab_t.shape[1]
    grid = ((num_emb // 4 + _BJ - 1) // _BJ,)
    return pl.pallas_call(
        _tr_body,
        grid=grid,
        in_specs=[pl.BlockSpec((D, 4 * _BJ), lambda j: (0, j))],
        out_specs=pl.BlockSpec((_BJ, 128), lambda j: (j, 0)),
        out_shape=jax.ShapeDtypeStruct((num_emb // 4, 128), jnp.float32),
    )(tab_t)


def kernel(token_ids, embedding_matrix):
    ids_flat = token_ids.reshape(-1)
    tab_lin = _tc_transpose(embedding_matrix.T)
    out5 = _sc_gather(ids_flat, tab_lin.reshape(embedding_matrix.shape))
    return out5.transpose(2, 4, 0, 1, 3).reshape(NI, NJ, D)


# speedup vs baseline: 2.8544x; 1.0171x over previous
import functools

import jax
import jax.numpy as jnp
from jax import lax
from jax.experimental import pallas as pl
from jax.experimental.pallas import tpu as pltpu
from jax.experimental.pallas import tpu_sc as plsc

NUM_CORES = 2
NUM_SUBCORES = 16
NUM_WORKERS = NUM_CORES * NUM_SUBCORES

NI = 16384
NJ = 50
D = 32
SUBI = 16
CHUNK = SUBI * NJ
I_PER_W = NI // NUM_WORKERS
N_CHUNKS = I_PER_W // SUBI


def _gather_body(ids_hbm, tab_hbm, out_hbm, idx_v, rows_v, w_v, isem, gsem,
                 ssem):
    wid = lax.axis_index("s") * NUM_CORES + lax.axis_index("c")
    i_base = wid * I_PER_W

    lane = lax.iota(jnp.int32, 16)
    dhalf = []
    for d0 in (0, 16):
        d_all = lane + d0
        dhalf.append((lax.shift_right_logical(d_all, 3),
                      lax.bitwise_and(d_all, 7)))

    def start_idx(g, b):
        pltpu.async_copy(
            ids_hbm.at[pl.ds((i_base + g * SUBI) * NJ, CHUNK)], idx_v.at[b],
            isem[b])

    def wait_idx(g, b):
        pltpu.make_async_copy(
            ids_hbm.at[pl.ds((i_base + g * SUBI) * NJ, CHUNK)], idx_v.at[b],
            isem[b]).wait()

    def start_gather(b):
        pltpu.async_copy(tab_hbm.at[idx_v.at[b]], rows_v.at[b], gsem[b])

    def wait_gather(b):
        pltpu.make_async_copy(tab_hbm.at[idx_v.at[b]], rows_v.at[b],
                              gsem[b]).wait()

    def _store_pair(g, b, j):
        i0 = i_base + g * SUBI
        return (w_v.at[b, j, :, :, pl.ds(0, SUBI)],
                out_hbm.at[j, :, i0 // 128, :, pl.ds(i0 % 128, SUBI)])

    def start_store(g, b):
        def per_j(j, carry):
            src, dst = _store_pair(g, b, j)
            pltpu.async_copy(src, dst, ssem[b])
            return carry
        lax.fori_loop(0, NJ, per_j, 0, unroll=False)

    def wait_store(g, b):
        def per_j(j, carry):
            src, dst = _store_pair(g, b, j)
            pltpu.make_async_copy(src, dst, ssem[b]).wait()
            return carry
        lax.fori_loop(0, NJ, per_j, 0, unroll=False)

    def transpose_chunk(b):
        rows = rows_v.at[b]
        w = w_v.at[b]

        def per_iloc(i_loc, c0):
            iv = jnp.full((16,), i_loc, jnp.int32)
            kbase = i_loc * NJ

            def per_j(j, c1):
                wj = w.at[j]
                k = kbase + j
                for h, (dhi, dlo) in enumerate(dhalf):
                    vals = rows[k, pl.ds(h * 16, 16)]
                    plsc.store_scatter(wj, [dhi, dlo, iv], vals)
                return c1

            lax.fori_loop(0, NJ, per_j, 0, unroll=10)
            return c0

        lax.fori_loop(0, SUBI, per_iloc, 0, unroll=False)

    start_idx(0, 0)
    start_idx(1, 1)
    wait_idx(0, 0)
    start_gather(0)

    def pair(gg, carry):
        g0 = gg * 2
        for b in range(2):
            g = g0 + b
            nb = 1 - b
            wait_gather(b)
            @pl.when(g + 1 < N_CHUNKS)
            def _():
                wait_idx(g + 1, nb)
                start_gather(nb)

            @pl.when(g >= 2)
            def _():
                wait_store(g - 2, b)

            transpose_chunk(b)
            start_store(g, b)

            @pl.when(g + 2 < N_CHUNKS)
            def _():
                start_idx(g + 2, b)
        return carry

    lax.fori_loop(0, N_CHUNKS // 2, pair, 0, unroll=False)

    for g in range(N_CHUNKS - 2, N_CHUNKS):
        wait_store(g, g % 2)


@jax.jit
def _sc_gather(ids_flat, table):
    mesh = plsc.VectorSubcoreMesh(
        core_axis_name="c", subcore_axis_name="s",
        num_cores=NUM_CORES, num_subcores=NUM_SUBCORES)
    return pl.kernel(
        _gather_body,
        out_type=jax.ShapeDtypeStruct((NJ, D // 8, NI // 128, 8, 128),
                                      jnp.float32),
        mesh=mesh,
        scratch_types=[
            pltpu.VMEM((2, CHUNK), jnp.int32),
            pltpu.VMEM((2, CHUNK, D), jnp.float32),
            pltpu.VMEM((2, NJ, D // 8, 8, SUBI + 1), jnp.float32),
            [pltpu.SemaphoreType.DMA] * 2,
            [pltpu.SemaphoreType.DMA] * 2,
            [pltpu.SemaphoreType.DMA] * 2,
        ],
        compiler_params=pltpu.CompilerParams(use_tc_tiling_on_sc=False,
                                             needs_layout_passes=False,
                                             disable_bounds_checks=True),
    )(ids_flat, table)


_BJ = 4096


def _tr_body(in_ref, out_ref):
    y = in_ref[...].T.reshape(_BJ, 4, 32)
    for c in range(4):
        out_ref[:, c * 32:(c + 1) * 32] = y[:, c, :]


@jax.jit
def _tc_transpose(tab_t):
    num_emb = tab_t.shape[1]
    grid = ((num_emb // 4 + _BJ - 1) // _BJ,)
    return pl.pallas_call(
        _tr_body,
        grid=grid,
        in_specs=[pl.BlockSpec((D, 4 * _BJ), lambda j: (0, j))],
        out_specs=pl.BlockSpec((_BJ, 128), lambda j: (j, 0)),
        out_shape=jax.ShapeDtypeStruct((num_emb // 4, 128), jnp.float32),
    )(tab_t)


def kernel(token_ids, embedding_matrix):
    ids_flat = token_ids.reshape(-1)
    tab_lin = _tc_transpose(embedding_matrix.T)
    out5 = _sc_gather(ids_flat, tab_lin.reshape(embedding_matrix.shape))
    return out5.transpose(2, 4, 0, 1, 3).reshape(NI, NJ, D)


# speedup vs baseline: 2.8654x; 1.0039x over previous
import functools

import jax
import jax.numpy as jnp
from jax import lax
from jax.experimental import pallas as pl
from jax.experimental.pallas import tpu as pltpu
from jax.experimental.pallas import tpu_sc as plsc

NUM_CORES = 2
NUM_SUBCORES = 16
NUM_WORKERS = NUM_CORES * NUM_SUBCORES

NI = 16384
NJ = 50
D = 32
SUBI = 16
CHUNK = SUBI * NJ
I_PER_W = NI // NUM_WORKERS
N_CHUNKS = I_PER_W // SUBI


def _gather_body(ids_hbm, tab_hbm, out_hbm, idx_v, rows_v, w_v, isem, gsem,
                 ssem):
    wid = lax.axis_index("s") * NUM_CORES + lax.axis_index("c")
    i_base = wid * I_PER_W

    lane = lax.iota(jnp.int32, 16)
    dhalf = []
    for d0 in (0, 16):
        d_all = lane + d0
        dhalf.append((lax.shift_right_logical(d_all, 3),
                      lax.bitwise_and(d_all, 7)))

    def start_idx(g, b):
        pltpu.async_copy(
            ids_hbm.at[pl.ds((i_base + g * SUBI) * NJ, CHUNK)], idx_v.at[b],
            isem[b])

    def wait_idx(g, b):
        pltpu.make_async_copy(
            ids_hbm.at[pl.ds((i_base + g * SUBI) * NJ, CHUNK)], idx_v.at[b],
            isem[b]).wait()

    def start_gather(b):
        pltpu.async_copy(tab_hbm.at[idx_v.at[b]], rows_v.at[b], gsem[b])

    def wait_gather(b):
        pltpu.make_async_copy(tab_hbm.at[idx_v.at[b]], rows_v.at[b],
                              gsem[b]).wait()

    def _store_pair(g, b, j):
        i0 = i_base + g * SUBI
        return (w_v.at[b, j, :, :, pl.ds(0, SUBI)],
                out_hbm.at[j, :, i0 // 128, :, pl.ds(i0 % 128, SUBI)])

    def start_store(g, b):
        def per_j(j, carry):
            src, dst = _store_pair(g, b, j)
            pltpu.async_copy(src, dst, ssem[b])
            return carry
        lax.fori_loop(0, NJ, per_j, 0, unroll=False)

    def wait_store(g, b):
        def per_j(j, carry):
            src, dst = _store_pair(g, b, j)
            pltpu.make_async_copy(src, dst, ssem[b]).wait()
            return carry
        lax.fori_loop(0, NJ, per_j, 0, unroll=False)

    def transpose_chunk(b):
        rows = rows_v.at[b]
        w = w_v.at[b]

        def per_iloc(i_loc, c0):
            iv = jnp.full((16,), i_loc, jnp.int32)
            kbase = i_loc * NJ

            def per_j(j, c1):
                wj = w.at[j]
                k = kbase + j
                for h, (dhi, dlo) in enumerate(dhalf):
                    vals = rows[k, pl.ds(h * 16, 16)]
                    plsc.store_scatter(wj, [dhi, dlo, iv], vals)
                return c1

            lax.fori_loop(0, NJ, per_j, 0, unroll=25)
            return c0

        lax.fori_loop(0, SUBI, per_iloc, 0, unroll=False)

    start_idx(0, 0)
    start_idx(1, 1)
    wait_idx(0, 0)
    start_gather(0)

    def pair(gg, carry):
        g0 = gg * 2
        for b in range(2):
            g = g0 + b
            nb = 1 - b
            wait_gather(b)
            @pl.when(g + 1 < N_CHUNKS)
            def _():
                wait_idx(g + 1, nb)
                start_gather(nb)

            @pl.when(g >= 2)
            def _():
                wait_store(g - 2, b)

            transpose_chunk(b)
            start_store(g, b)

            @pl.when(g + 2 < N_CHUNKS)
            def _():
                start_idx(g + 2, b)
        return carry

    lax.fori_loop(0, N_CHUNKS // 2, pair, 0, unroll=False)

    for g in range(N_CHUNKS - 2, N_CHUNKS):
        wait_store(g, g % 2)


@jax.jit
def _sc_gather(ids_flat, table):
    mesh = plsc.VectorSubcoreMesh(
        core_axis_name="c", subcore_axis_name="s",
        num_cores=NUM_CORES, num_subcores=NUM_SUBCORES)
    return pl.kernel(
        _gather_body,
        out_type=jax.ShapeDtypeStruct((NJ, D // 8, NI // 128, 8, 128),
                                      jnp.float32),
        mesh=mesh,
        scratch_types=[
            pltpu.VMEM((2, CHUNK), jnp.int32),
            pltpu.VMEM((2, CHUNK, D), jnp.float32),
            pltpu.VMEM((2, NJ, D // 8, 8, SUBI + 1), jnp.float32),
            [pltpu.SemaphoreType.DMA] * 2,
            [pltpu.SemaphoreType.DMA] * 2,
            [pltpu.SemaphoreType.DMA] * 2,
        ],
        compiler_params=pltpu.CompilerParams(use_tc_tiling_on_sc=False,
                                             needs_layout_passes=False,
                                             disable_bounds_checks=True),
    )(ids_flat, table)


_BJ = 4096


def _tr_body(in_ref, out_ref):
    y = in_ref[...].T.reshape(_BJ, 4, 32)
    for c in range(4):
        out_ref[:, c * 32:(c + 1) * 32] = y[:, c, :]


@jax.jit
def _tc_transpose(tab_t):
    num_emb = tab_t.shape[1]
    grid = ((num_emb // 4 + _BJ - 1) // _BJ,)
    return pl.pallas_call(
        _tr_body,
        grid=grid,
        in_specs=[pl.BlockSpec((D, 4 * _BJ), lambda j: (0, j))],
        out_specs=pl.BlockSpec((_BJ, 128), lambda j: (j, 0)),
        out_shape=jax.ShapeDtypeStruct((num_emb // 4, 128), jnp.float32),
    )(tab_t)


def kernel(token_ids, embedding_matrix):
    ids_flat = token_ids.reshape(-1)
    tab_lin = _tc_transpose(embedding_matrix.T)
    out5 = _sc_gather(ids_flat, tab_lin.reshape(embedding_matrix.shape))
    return out5.transpose(2, 4, 0, 1, 3).reshape(NI, NJ, D)


# speedup vs baseline: 2.8677x; 1.0008x over previous
import jax
import jax.numpy as jnp
from jax import lax
from jax.experimental import pallas as pl
from jax.experimental.pallas import tpu as pltpu
from jax.experimental.pallas import tpu_sc as plsc

NUM_CORES = 2
NUM_SUBCORES = 16
NUM_WORKERS = NUM_CORES * NUM_SUBCORES

NI = 16384
NJ = 50
D = 32
SUBI = 16
CHUNK = SUBI * NJ
I_PER_W = NI // NUM_WORKERS
N_CHUNKS = I_PER_W // SUBI


def _gather_body(ids_hbm, tab_hbm, out_hbm, idx_v, rows_v, w_v, isem, gsem,
                 ssem):
    wid = lax.axis_index("s") * NUM_CORES + lax.axis_index("c")
    i_base = wid * I_PER_W

    lane = lax.iota(jnp.int32, 16)
    dhalf = []
    for d0 in (0, 16):
        d_all = lane + d0
        dhalf.append((lax.shift_right_logical(d_all, 3),
                      lax.bitwise_and(d_all, 7)))

    def start_idx(g, b):
        pltpu.async_copy(
            ids_hbm.at[pl.ds((i_base + g * SUBI) * NJ, CHUNK)], idx_v.at[b],
            isem[b])

    def wait_idx(g, b):
        pltpu.make_async_copy(
            ids_hbm.at[pl.ds((i_base + g * SUBI) * NJ, CHUNK)], idx_v.at[b],
            isem[b]).wait()

    def start_gather(b):
        pltpu.async_copy(tab_hbm.at[idx_v.at[b]], rows_v.at[b], gsem[b])

    def wait_gather(b):
        pltpu.make_async_copy(tab_hbm.at[idx_v.at[b]], rows_v.at[b],
                              gsem[b]).wait()

    def _store_pair(g, b, j):
        i0 = i_base + g * SUBI
        return (w_v.at[b, j, :, :, pl.ds(0, SUBI)],
                out_hbm.at[j, :, i0 // 128, :, pl.ds(i0 % 128, SUBI)])

    def start_store(g, b):
        def per_j(j, carry):
            src, dst = _store_pair(g, b, j)
            pltpu.async_copy(src, dst, ssem[b])
            return carry
        lax.fori_loop(0, NJ, per_j, 0, unroll=False)

    def wait_store(g, b):
        def per_j(j, carry):
            src, dst = _store_pair(g, b, j)
            pltpu.make_async_copy(src, dst, ssem[b]).wait()
            return carry
        lax.fori_loop(0, NJ, per_j, 0, unroll=False)

    def transpose_chunk(b):
        rows = rows_v.at[b]
        w = w_v.at[b]

        def per_iloc(i_loc, c0):
            iv = jnp.full((16,), i_loc, jnp.int32)
            kbase = i_loc * NJ

            def per_j(j, c1):
                wj = w.at[j]
                k = kbase + j
                for h, (dhi, dlo) in enumerate(dhalf):
                    vals = rows[k, pl.ds(h * 16, 16)]
                    plsc.store_scatter(wj, [dhi, dlo, iv], vals)
                return c1

            lax.fori_loop(0, NJ, per_j, 0, unroll=25)
            return c0

        lax.fori_loop(0, SUBI, per_iloc, 0, unroll=False)

    start_idx(0, 0)
    start_idx(1, 1)
    wait_idx(0, 0)
    start_gather(0)

    def pair(gg, carry):
        g0 = gg * 2
        for b in range(2):
            g = g0 + b
            nb = 1 - b
            wait_gather(b)
            @pl.when(g + 1 < N_CHUNKS)
            def _():
                wait_idx(g + 1, nb)
                start_gather(nb)

            @pl.when(g >= 2)
            def _():
                wait_store(g - 2, b)

            transpose_chunk(b)
            start_store(g, b)

            @pl.when(g + 2 < N_CHUNKS)
            def _():
                start_idx(g + 2, b)
        return carry

    lax.fori_loop(0, N_CHUNKS // 2, pair, 0, unroll=False)

    for g in range(N_CHUNKS - 2, N_CHUNKS):
        wait_store(g, g % 2)


@jax.jit
def _sc_gather(ids_flat, table):
    mesh = plsc.VectorSubcoreMesh(
        core_axis_name="c", subcore_axis_name="s",
        num_cores=NUM_CORES, num_subcores=NUM_SUBCORES)
    return pl.kernel(
        _gather_body,
        out_type=jax.ShapeDtypeStruct((NJ, D // 8, NI // 128, 8, 128),
                                      jnp.float32),
        mesh=mesh,
        scratch_types=[
            pltpu.VMEM((2, CHUNK), jnp.int32),
            pltpu.VMEM((2, CHUNK, D), jnp.float32),
            pltpu.VMEM((2, NJ, D // 8, 8, SUBI + 1), jnp.float32),
            [pltpu.SemaphoreType.DMA] * 2,
            [pltpu.SemaphoreType.DMA] * 2,
            [pltpu.SemaphoreType.DMA] * 2,
        ],
        compiler_params=pltpu.CompilerParams(use_tc_tiling_on_sc=False,
                                             needs_layout_passes=False,
                                             disable_bounds_checks=True),
    )(ids_flat, table)


_BJ = 4096


def _tr_body(in_ref, out_ref):
    y = in_ref[...].T.reshape(_BJ, 4, 32)
    for c in range(4):
        out_ref[:, c * 32:(c + 1) * 32] = y[:, c, :]


@jax.jit
def _tc_transpose(tab_t):
    num_emb = tab_t.shape[1]
    grid = ((num_emb // 4 + _BJ - 1) // _BJ,)
    return pl.pallas_call(
        _tr_body,
        grid=grid,
        in_specs=[pl.BlockSpec((D, 4 * _BJ), lambda j: (0, j))],
        out_specs=pl.BlockSpec((_BJ, 128), lambda j: (j, 0)),
        out_shape=jax.ShapeDtypeStruct((num_emb // 4, 128), jnp.float32),
    )(tab_t)


def kernel(token_ids, embedding_matrix):
    ids_flat = token_ids.reshape(-1)
    tab_lin = _tc_transpose(embedding_matrix.T)
    out5 = _sc_gather(ids_flat, tab_lin.reshape(embedding_matrix.shape))
    return out5.transpose(2, 4, 0, 1, 3).reshape(NI, NJ, D)
